# Initial kernel scaffold; baseline (speedup 1.0000x reference)
import jax
import jax.numpy as jnp
from jax.experimental import pallas as pl


def kernel(x_alpha, x_beta, x_theta, x_gamma, params, edge_index_alpha, edge_index_beta, edge_index_theta, edge_index_gamma, batch_alpha, batch_beta, batch_theta, batch_gamma):
    raise NotImplementedError("write your pallas kernel here")


# speedup vs baseline: 50.4747x; 50.4747x over previous
import functools

import jax
import jax.numpy as jnp
from jax import lax
from jax.experimental import pallas as pl
from jax.experimental.pallas import tpu as pltpu
from jax.experimental.pallas import tpu_sc as plsc

N = 10000
NP = 10240
E = 320000
EP = 331776
F = 128
HD = 8
CH = 8
NB = 4
TPW = EP // 32
NCHUNK = TPW // 128
NEG = -1e9
MBR = 2048


def _prep_body(x_ref, g_ref, b_ref, w_ref, asf_ref, adf_ref, attab_ref, htab_ref):
    x = x_ref[0]
    m = jnp.mean(x, axis=0)
    v = jnp.mean((x - m) ** 2, axis=0)
    xn = (x - m) / jnp.sqrt(v + 1e-5) * g_ref[0, 0] + b_ref[0, 0]
    h = jnp.dot(xn, w_ref[0], preferred_element_type=jnp.float32)
    r_i = lax.broadcasted_iota(jnp.int32, (HD * CH, HD), 0)
    c_i = lax.broadcasted_iota(jnp.int32, (HD * CH, HD), 1)
    sel = (r_i // CH) == c_i
    abd_s = jnp.where(sel, asf_ref[0, 0].reshape(HD * CH, 1), 0.0)
    abd_d = jnp.where(sel, adf_ref[0, 0].reshape(HD * CH, 1), 0.0)
    as_ = jnp.dot(h, abd_s, preferred_element_type=jnp.float32)
    ad_ = jnp.dot(h, abd_d, preferred_element_type=jnp.float32)
    attab_ref[0, :N, :] = jnp.concatenate([as_, ad_], axis=1)
    attab_ref[0, N:, :] = jnp.full((NP - N, 2 * HD), NEG, jnp.float32)
    htab_ref[0, :N, :] = h
    htab_ref[0, N:, :] = jnp.zeros((NP - N, HD * CH), jnp.float32)


def _prep_call(x_all, g_all, b_all, w_all, asf_all, adf_all):
    return pl.pallas_call(
        _prep_body,
        grid=(NB,),
        in_specs=[
            pl.BlockSpec((1, N, F), lambda b: (b, 0, 0)),
            pl.BlockSpec((1, 1, F), lambda b: (b, 0, 0)),
            pl.BlockSpec((1, 1, F), lambda b: (b, 0, 0)),
            pl.BlockSpec((1, F, HD * CH), lambda b: (b, 0, 0)),
            pl.BlockSpec((1, 1, HD * CH), lambda b: (b, 0, 0)),
            pl.BlockSpec((1, 1, HD * CH), lambda b: (b, 0, 0)),
        ],
        out_specs=[
            pl.BlockSpec((1, NP, 2 * HD), lambda b: (b, 0, 0)),
            pl.BlockSpec((1, NP, HD * CH), lambda b: (b, 0, 0)),
        ],
        out_shape=[
            jax.ShapeDtypeStruct((NB, NP, 2 * HD), jnp.float32),
            jax.ShapeDtypeStruct((NB, NP, HD * CH), jnp.float32),
        ],
    )(x_all, g_all, b_all, w_all, asf_all, adf_all)


def _vgather(v, idx):
    return lax.gather(
        v, idx[:, None],
        lax.GatherDimensionNumbers(
            offset_dims=(), collapsed_slice_dims=(0,), start_index_map=(0,)),
        (1,), mode=lax.GatherScatterMode.PROMISE_IN_BOUNDS)


_ROT8 = tuple((i + 8) % 16 for i in range(16))


def _edge1_body(attab, htab, srcg, dstg, z16, z64,
                den_out, agg_out,
                sidx, didx, draw, gs, gd, gh, exb, sem,
                den_acc, agg_acc):
    c = lax.axis_index("c")
    s = lax.axis_index("s")
    w = s * 2 + c
    rowbase = s * (NP // 16)
    nrows = NP // 16
    lane = lax.iota(jnp.int32, 16)
    rot8 = (lane + 8) & 15
    hsel = [2 * j + (lane >> 3) for j in range(4)]

    if True:
        def zero_accs():
            pltpu.sync_copy(z16.at[pl.ds(rowbase, nrows)],
                            den_acc.at[pl.ds(rowbase, nrows)])
            pltpu.sync_copy(z64.at[pl.ds(rowbase, nrows)],
                            agg_acc.at[pl.ds(rowbase, nrows)])

        def edge_fn(r, _):
            vs = gs[r, :]
            vd = gd[r, :]
            e = vs + _vgather(vd, rot8)
            e = jnp.where(e > 0.0, e, e * 0.2)
            ex = jnp.exp(e)
            exb[r, :] = ex
            for j in range(4):
                rep = _vgather(ex, hsel[j])
                gh[r, pl.ds(j * 16, 16)] = gh[r, pl.ds(j * 16, 16)] * rep
            return _

        def chunk_fn(b, j, _):
            base = b * EP + w * TPW + j * 128
            pltpu.sync_copy(srcg.at[pl.ds(base, 128)], sidx)
            pltpu.sync_copy(dstg.at[pl.ds(base, 128)], didx)
            cp1 = pltpu.async_copy(attab.at[sidx], gs, sem)
            cp2 = pltpu.async_copy(attab.at[didx], gd, sem)
            cp3 = pltpu.async_copy(htab.at[sidx], gh, sem)
            off = jnp.int32(b * NP)
            for t in range(8):
                draw[pl.ds(t * 16, 16)] = didx[pl.ds(t * 16, 16)] - off
            cp1.wait()
            cp2.wait()
            cp3.wait()
            lax.fori_loop(0, 128, edge_fn, None, unroll=2)
            pltpu.sync_copy(exb, den_acc.at[draw], add=True)
            pltpu.sync_copy(gh, agg_acc.at[draw], add=True)
            return _

        zero_accs()
        plsc.subcore_barrier()
        for b in range(NB):
            lax.fori_loop(0, NCHUNK, functools.partial(chunk_fn, b), None)
            plsc.subcore_barrier()
            pltpu.sync_copy(den_acc.at[pl.ds(rowbase, nrows)],
                            den_out.at[c, b, pl.ds(rowbase, nrows)])
            pltpu.sync_copy(agg_acc.at[pl.ds(rowbase, nrows)],
                            agg_out.at[c, b, pl.ds(rowbase, nrows)])
            if b < NB - 1:
                zero_accs()
            plsc.subcore_barrier()


def _edge1_call(attab_flat, htab_flat, srcg, dstg, z16, z64):
    mesh = plsc.VectorSubcoreMesh(core_axis_name="c", subcore_axis_name="s")
    fn = pl.kernel(
        _edge1_body,
        out_type=[
            jax.ShapeDtypeStruct((2, NB, NP, 2 * HD), jnp.float32),
            jax.ShapeDtypeStruct((2, NB, NP, HD * CH), jnp.float32),
        ],
        mesh=mesh,
        compiler_params=pltpu.CompilerParams(use_tc_tiling_on_sc=False),
        scratch_types=[
            pltpu.VMEM((128,), jnp.int32),
            pltpu.VMEM((128,), jnp.int32),
            pltpu.VMEM((128,), jnp.int32),
            pltpu.VMEM((128, 2 * HD), jnp.float32),
            pltpu.VMEM((128, 2 * HD), jnp.float32),
            pltpu.VMEM((128, HD * CH), jnp.float32),
            pltpu.VMEM((128, 2 * HD), jnp.float32),
            pltpu.SemaphoreType.DMA,
            pltpu.VMEM_SHARED((NP, 2 * HD), jnp.float32),
            pltpu.VMEM_SHARED((NP, HD * CH), jnp.float32),
        ],
    )
    return fn(attab_flat, htab_flat, srcg, dstg, z16, z64)


def _mid_body(d0_ref, d1_ref, a0_ref, a1_ref, attab_ref, b1_ref,
              g2_ref, b2_ref, w2_ref, a2s_ref, a2d_ref,
              tab2_ref, attntab_ref, stats_ref):
    ph = pl.program_id(1)
    blk = pl.program_id(2)
    den = d0_ref[0, 0] + d1_ref[0, 0]
    dsum = den[:, 0:HD]
    num = a0_ref[0, 0] + a1_ref[0, 0]
    r_i = lax.broadcasted_iota(jnp.int32, (HD, HD * CH), 0)
    c_i = lax.broadcasted_iota(jnp.int32, (HD, HD * CH), 1)
    rep = jnp.where((c_i // CH) == r_i, 1.0, 0.0)
    drep = jnp.dot(dsum, rep, preferred_element_type=jnp.float32)
    agg = num / (drep + 1e-16)
    x1 = jnp.maximum(agg + b1_ref[0, 0], 0.0)
    grow = blk * MBR + lax.broadcasted_iota(jnp.int32, (MBR, 1), 0)
    live = grow < N

    @pl.when((ph == 0) & (blk == 0))
    def _init():
        stats_ref[...] = jnp.zeros((2, HD * CH), jnp.float32)

    @pl.when(ph == 0)
    def _accum():
        x1m = jnp.where(live, x1, 0.0)
        stats_ref[0, :] += jnp.sum(x1m, axis=0)
        stats_ref[1, :] += jnp.sum(x1m * x1m, axis=0)

    m2 = stats_ref[0, :] / N
    v2 = stats_ref[1, :] / N - m2 * m2
    xn2 = (x1 - m2) / jnp.sqrt(v2 + 1e-5) * g2_ref[0, 0] + b2_ref[0, 0]
    h2 = jnp.dot(xn2, w2_ref[0], preferred_element_type=jnp.float32)
    pi = lax.broadcasted_iota(jnp.int32, (8, 16), 0)
    pj = lax.broadcasted_iota(jnp.int32, (8, 16), 1)
    P = (a2s_ref[0, 0].reshape(8, 1) * (pj == 0)
         + a2d_ref[0, 0].reshape(8, 1) * (pj == 1)
         + jnp.where(pj == pi + 2, 1.0, 0.0))
    tab2 = jnp.dot(h2, P, preferred_element_type=jnp.float32)
    di = lax.broadcasted_iota(jnp.int32, (MBR, 16), 1)
    dummy = jnp.where(di < 2, NEG, 0.0)
    tab2_ref[0] = jnp.where(live, tab2, dummy)
    attntab_ref[0] = jnp.concatenate(
        [attab_ref[0][:, HD:2 * HD], dsum], axis=1)


def _mid_call(den_out, agg_out, attab_all, b1_all, g2_all, b2_all,
              w2p_all, a2s_all, a2d_all):
    nbk = NP // MBR
    return pl.pallas_call(
        _mid_body,
        grid=(NB, 2, nbk),
        in_specs=[
            pl.BlockSpec((1, 1, MBR, 16), lambda b, p, k: (0, b, k, 0)),
            pl.BlockSpec((1, 1, MBR, 16), lambda b, p, k: (1, b, k, 0)),
            pl.BlockSpec((1, 1, MBR, 64), lambda b, p, k: (0, b, k, 0)),
            pl.BlockSpec((1, 1, MBR, 64), lambda b, p, k: (1, b, k, 0)),
            pl.BlockSpec((1, MBR, 16), lambda b, p, k: (b, k, 0)),
            pl.BlockSpec((1, 1, 64), lambda b, p, k: (b, 0, 0)),
            pl.BlockSpec((1, 1, 64), lambda b, p, k: (b, 0, 0)),
            pl.BlockSpec((1, 1, 64), lambda b, p, k: (b, 0, 0)),
            pl.BlockSpec((1, 64, 8), lambda b, p, k: (b, 0, 0)),
            pl.BlockSpec((1, 1, 8), lambda b, p, k: (b, 0, 0)),
            pl.BlockSpec((1, 1, 8), lambda b, p, k: (b, 0, 0)),
        ],
        out_specs=[
            pl.BlockSpec((1, MBR, 16), lambda b, p, k: (b, k, 0)),
            pl.BlockSpec((1, MBR, 16), lambda b, p, k: (b, k, 0)),
        ],
        out_shape=[
            jax.ShapeDtypeStruct((NB, NP, 16), jnp.float32),
            jax.ShapeDtypeStruct((NB, NP, 16), jnp.float32),
        ],
        scratch_shapes=[pltpu.VMEM((2, HD * CH), jnp.float32)],
        compiler_params=pltpu.CompilerParams(
            dimension_semantics=("arbitrary", "arbitrary", "arbitrary")),
    )(den_out, den_out, agg_out, agg_out, attab_all, b1_all,
      g2_all, b2_all, w2p_all, a2s_all, a2d_all)


def _edge2_body(tab2, attab, attntab, srcg, dstg, z16,
                acc2_out, attn_out,
                sidx, didx, gs, gd, vals, sem, acc2):
    c = lax.axis_index("c")
    s = lax.axis_index("s")
    w = s * 2 + c
    arows = NB * NP // 16
    lane = lax.iota(jnp.int32, 16)
    sel1 = lane * 0 + 1
    sel0 = lane * 0
    hsel2 = jnp.where((lane >= 8) & (lane < 12), lane - 6, 6)
    one0 = jnp.where(lane == 0, 1.0, 0.0)
    denh = 8 + (lane & 7)

    if True:
        pltpu.sync_copy(z16.at[pl.ds(0, arows)],
                        acc2.at[pl.ds(s * arows, arows)])
        plsc.subcore_barrier()

        def edge_fn(r, _):
            vs = gs[r, :]
            vd = gd[r, :]
            t = vs + _vgather(vd, sel1)
            e2 = _vgather(t, sel0)
            e2 = jnp.where(e2 > 0.0, e2, e2 * 0.2)
            ex2 = jnp.exp(e2)
            vals[r, :] = ex2 * (_vgather(vs, hsel2) + one0)
            return _

        def chunk_fn(b, j, _):
            base = b * EP + w * TPW + j * 128
            pltpu.sync_copy(srcg.at[pl.ds(base, 128)], sidx)
            pltpu.sync_copy(dstg.at[pl.ds(base, 128)], didx)
            cp1 = pltpu.async_copy(tab2.at[sidx], gs, sem)
            cp2 = pltpu.async_copy(tab2.at[didx], gd, sem)
            cp1.wait()
            cp2.wait()
            lax.fori_loop(0, 128, edge_fn, None, unroll=2)
            pltpu.sync_copy(vals, acc2.at[didx], add=True)
            return _

        for b in range(NB):
            lax.fori_loop(0, NCHUNK, functools.partial(chunk_fn, b), None)
        plsc.subcore_barrier()
        pltpu.sync_copy(acc2.at[pl.ds(s * arows, arows)],
                        acc2_out.at[c, pl.ds(s * arows, arows)])

        def attn_edge(r, _):
            vs = gs[r, :]
            vd = gd[r, :]
            e = vs + vd
            e = jnp.where(e > 0.0, e, e * 0.2)
            ex = jnp.exp(e)
            denv = _vgather(vd, denh)
            vals[r, :] = ex / (denv + 1e-16)
            return _

        def attn_chunk(j, _):
            base = w * TPW + j * 128
            pltpu.sync_copy(srcg.at[pl.ds(base, 128)], sidx)
            pltpu.sync_copy(dstg.at[pl.ds(base, 128)], didx)
            cp1 = pltpu.async_copy(attab.at[sidx], gs, sem)
            cp2 = pltpu.async_copy(attntab.at[didx], gd, sem)
            cp1.wait()
            cp2.wait()
            lax.fori_loop(0, 128, attn_edge, None, unroll=2)
            pltpu.sync_copy(vals, attn_out.at[pl.ds(base, 128)])
            return _

        lax.fori_loop(0, NCHUNK, attn_chunk, None)


def _edge2_call(tab2_flat, attab_flat, attntab, srcg, dstg, z16):
    mesh = plsc.VectorSubcoreMesh(core_axis_name="c", subcore_axis_name="s")
    fn = pl.kernel(
        _edge2_body,
        out_type=[
            jax.ShapeDtypeStruct((2, NB * NP, 16), jnp.float32),
            jax.ShapeDtypeStruct((EP, 16), jnp.float32),
        ],
        mesh=mesh,
        compiler_params=pltpu.CompilerParams(use_tc_tiling_on_sc=False),
        scratch_types=[
            pltpu.VMEM((128,), jnp.int32),
            pltpu.VMEM((128,), jnp.int32),
            pltpu.VMEM((128, 16), jnp.float32),
            pltpu.VMEM((128, 16), jnp.float32),
            pltpu.VMEM((128, 16), jnp.float32),
            pltpu.SemaphoreType.DMA,
            pltpu.VMEM_SHARED((NB * NP, 16), jnp.float32),
        ],
    )
    return fn(tab2_flat, attab_flat, attntab, srcg, dstg, z16)


def _pool_body(a20_ref, a21_ref, bb2_ref, batch_ref, ls_ref):
    A = a20_ref[0] + a21_ref[0]
    bcast = jnp.where(lax.broadcasted_iota(jnp.int32, (8, 8), 0) == 0, 1.0, 0.0)
    d = jnp.dot(A[:, 0:8], bcast, preferred_element_type=jnp.float32)
    x2 = A[:, 8:16] / (d + 1e-16) + bb2_ref[0, 0]
    x2 = jnp.where(x2 > 0.0, x2, jnp.exp(jnp.minimum(x2, 0.0)) - 1.0)
    gsel = lax.broadcasted_iota(jnp.int32, (1, 64), 1)
    oh = (batch_ref[0, 0].reshape(NP, 1) == gsel).astype(jnp.float32)
    ssum = lax.dot_general(oh, x2, (((0,), (0,)), ((), ())),
                           preferred_element_type=jnp.float32)
    cnt = jnp.sum(oh, axis=0).reshape(64, 1)
    pooled = ssum / jnp.maximum(cnt, 1.0)
    lane8 = lax.broadcasted_iota(jnp.int32, (64, 8), 1)
    mask = lane8 < 4
    mx = jnp.max(jnp.where(mask, pooled, -jnp.inf), axis=1, keepdims=True)
    se = jnp.sum(jnp.where(mask, jnp.exp(pooled - mx), 0.0),
                 axis=1, keepdims=True)
    ls_ref[0] = pooled - mx - jnp.log(se)


def _fuse_body(ls_ref, wf_ref, bf_ref, out_ref):
    c32 = jnp.concatenate([ls_ref[b] for b in range(NB)], axis=1)
    si = lax.broadcasted_iota(jnp.int32, (32, 16), 0)
    sj = lax.broadcasted_iota(jnp.int32, (32, 16), 1)
    S = jnp.where((sj == (si // 8) * 4 + (si % 8)) & (si % 8 < 4), 1.0, 0.0)
    xc = jnp.dot(c32, S, preferred_element_type=jnp.float32)
    res = jnp.dot(xc, wf_ref[...], preferred_element_type=jnp.float32) \
        + bf_ref[...]
    out_ref[...] = jnp.maximum(res[:, 0:4], 0.0)


def _final_call(a2_out, bb2_all, batch_all, wfp, bfp):
    ls = pl.pallas_call(
        _pool_body,
        grid=(NB,),
        in_specs=[
            pl.BlockSpec((1, NP, 16), lambda b: (b, 0, 0)),
            pl.BlockSpec((1, NP, 16), lambda b: (b, 0, 0)),
            pl.BlockSpec((1, 1, 8), lambda b: (b, 0, 0)),
            pl.BlockSpec((1, 1, NP), lambda b: (b, 0, 0)),
        ],
        out_specs=pl.BlockSpec((1, 64, 8), lambda b: (b, 0, 0)),
        out_shape=jax.ShapeDtypeStruct((NB, 64, 8), jnp.float32),
    )(a2_out[0].reshape(NB, NP, 16), a2_out[1].reshape(NB, NP, 16),
      bb2_all, batch_all)
    return pl.pallas_call(
        _fuse_body,
        out_shape=jax.ShapeDtypeStruct((64, 4), jnp.float32),
    )(ls, wfp, bfp)


def kernel(x_alpha, x_beta, x_theta, x_gamma, params,
           edge_index_alpha, edge_index_beta, edge_index_theta,
           edge_index_gamma, batch_alpha, batch_beta, batch_theta,
           batch_gamma):
    bands = ['alpha', 'beta', 'theta', 'gamma']
    xs = [x_alpha, x_beta, x_theta, x_gamma]
    eis = [edge_index_alpha, edge_index_beta, edge_index_theta,
           edge_index_gamma]
    bts = [batch_alpha, batch_beta, batch_theta, batch_gamma]

    x_all = jnp.stack(xs)
    g_all = jnp.stack([params[b]['bn1_g'] for b in bands]).reshape(NB, 1, F)
    b_all = jnp.stack([params[b]['bn1_b'] for b in bands]).reshape(NB, 1, F)
    w_all = jnp.stack([params[b]['conv1']['W'] for b in bands])
    asf_all = jnp.stack([params[b]['conv1']['as'].reshape(1, -1) for b in bands])
    adf_all = jnp.stack([params[b]['conv1']['ad'].reshape(1, -1) for b in bands])
    b1_all = jnp.stack([params[b]['conv1']['b'] for b in bands]).reshape(NB, 1, 64)
    g2_all = jnp.stack([params[b]['bn2_g'] for b in bands]).reshape(NB, 1, 64)
    b2_all = jnp.stack([params[b]['bn2_b'] for b in bands]).reshape(NB, 1, 64)
    w2p_all = jnp.stack([
        jnp.pad(params[b]['conv2']['W'], ((0, 0), (0, 4))) for b in bands])
    a2s_all = jnp.stack([
        jnp.pad(params[b]['conv2']['as'].reshape(-1), (0, 4)) for b in bands]).reshape(NB, 1, 8)
    a2d_all = jnp.stack([
        jnp.pad(params[b]['conv2']['ad'].reshape(-1), (0, 4)) for b in bands]).reshape(NB, 1, 8)
    bb2_all = jnp.stack([
        jnp.pad(params[b]['conv2']['b'], (0, 4)) for b in bands]).reshape(NB, 1, 8)
    wfp = jnp.pad(params['Wf'], ((0, 0), (0, 4)))
    bfp = jnp.pad(params['bf'], (0, 4))

    loop = jnp.arange(N, dtype=jnp.int32)
    padv = jnp.full((EP - E - N,), N, dtype=jnp.int32)
    srcg = jnp.concatenate([
        jnp.concatenate([ei[0], loop, padv]) + b * NP
        for b, ei in enumerate(eis)])
    dstg = jnp.concatenate([
        jnp.concatenate([ei[1], loop, padv]) + b * NP
        for b, ei in enumerate(eis)])
    batch_all = jnp.stack([
        jnp.concatenate([bt, jnp.full((NP - N,), 64, jnp.int32)])
        for bt in bts]).reshape(NB, 1, NP)

    z16 = jnp.zeros((NP, 16), jnp.float32)
    z64 = jnp.zeros((NP, 64), jnp.float32)

    attab_all, htab_all = _prep_call(x_all, g_all, b_all, w_all,
                                     asf_all, adf_all)
    attab_flat = attab_all.reshape(NB * NP, 16)
    htab_flat = htab_all.reshape(NB * NP, 64)

    den_out, agg_out = _edge1_call(attab_flat, htab_flat, srcg, dstg,
                                   z16, z64)

    tab2_all, attntab_all = _mid_call(den_out, agg_out, attab_all, b1_all,
                                      g2_all, b2_all, w2p_all,
                                      a2s_all, a2d_all)

    acc2_out, attn_wide = _edge2_call(tab2_all.reshape(NB * NP, 16),
                                      attab_flat, attntab_all[0],
                                      srcg, dstg, z16)

    out = _final_call(acc2_out, bb2_all, batch_all, wfp, bfp)
    attn = attn_wide[:E + N, 0:8]
    return out, attn


# speedup vs baseline: 75.2067x; 1.4900x over previous
import functools

import jax
import jax.numpy as jnp
from jax import lax
from jax.experimental import pallas as pl
from jax.experimental.pallas import tpu as pltpu
from jax.experimental.pallas import tpu_sc as plsc

N = 10000
NP = 10240
E = 320000
EP = 331776
F = 128
HD = 8
CH = 8
NB = 4
TPW = EP // 32
NCHUNK = TPW // 128
NEG = -1e9
MBR = 2048


def _prep_body(x_ref, g_ref, b_ref, w_ref, asf_ref, adf_ref, attab_ref, htab_ref):
    x = x_ref[0]
    m = jnp.mean(x, axis=0)
    v = jnp.mean((x - m) ** 2, axis=0)
    xn = (x - m) / jnp.sqrt(v + 1e-5) * g_ref[0, 0] + b_ref[0, 0]
    h = jnp.dot(xn, w_ref[0], preferred_element_type=jnp.float32)
    r_i = lax.broadcasted_iota(jnp.int32, (HD * CH, HD), 0)
    c_i = lax.broadcasted_iota(jnp.int32, (HD * CH, HD), 1)
    sel = (r_i // CH) == c_i
    abd_s = jnp.where(sel, asf_ref[0, 0].reshape(HD * CH, 1), 0.0)
    abd_d = jnp.where(sel, adf_ref[0, 0].reshape(HD * CH, 1), 0.0)
    as_ = jnp.dot(h, abd_s, preferred_element_type=jnp.float32)
    ad_ = jnp.dot(h, abd_d, preferred_element_type=jnp.float32)
    attab_ref[0, :N, :] = jnp.concatenate([as_, ad_], axis=1)
    attab_ref[0, N:, :] = jnp.full((NP - N, 2 * HD), NEG, jnp.float32)
    htab_ref[0, :N, :] = h
    htab_ref[0, N:, :] = jnp.zeros((NP - N, HD * CH), jnp.float32)


def _prep_call(x_all, g_all, b_all, w_all, asf_all, adf_all):
    return pl.pallas_call(
        _prep_body,
        grid=(NB,),
        in_specs=[
            pl.BlockSpec((1, N, F), lambda b: (b, 0, 0)),
            pl.BlockSpec((1, 1, F), lambda b: (b, 0, 0)),
            pl.BlockSpec((1, 1, F), lambda b: (b, 0, 0)),
            pl.BlockSpec((1, F, HD * CH), lambda b: (b, 0, 0)),
            pl.BlockSpec((1, 1, HD * CH), lambda b: (b, 0, 0)),
            pl.BlockSpec((1, 1, HD * CH), lambda b: (b, 0, 0)),
        ],
        out_specs=[
            pl.BlockSpec((1, NP, 2 * HD), lambda b: (b, 0, 0)),
            pl.BlockSpec((1, NP, HD * CH), lambda b: (b, 0, 0)),
        ],
        out_shape=[
            jax.ShapeDtypeStruct((NB, NP, 2 * HD), jnp.float32),
            jax.ShapeDtypeStruct((NB, NP, HD * CH), jnp.float32),
        ],
    )(x_all, g_all, b_all, w_all, asf_all, adf_all)


def _vgather(v, idx):
    return lax.gather(
        v, idx[:, None],
        lax.GatherDimensionNumbers(
            offset_dims=(), collapsed_slice_dims=(0,), start_index_map=(0,)),
        (1,), mode=lax.GatherScatterMode.PROMISE_IN_BOUNDS)


_ROT8 = tuple((i + 8) % 16 for i in range(16))


def _edge1_body(attab, htab, srcg, dstg, z16, z64,
                den_out, agg_out,
                sidx, didx, draw, gs, gd, gh, exb, sem,
                den_acc, agg_acc):
    c = lax.axis_index("c")
    s = lax.axis_index("s")
    w = s * 2 + c
    rowbase = s * (NP // 16)
    nrows = NP // 16
    lane = lax.iota(jnp.int32, 16)
    rot8 = (lane + 8) & 15
    hsel = [2 * j + (lane >> 3) for j in range(4)]

    if True:
        def zero_accs():
            pltpu.sync_copy(z16.at[pl.ds(rowbase, nrows)],
                            den_acc.at[pl.ds(rowbase, nrows)])
            pltpu.sync_copy(z64.at[pl.ds(rowbase, nrows)],
                            agg_acc.at[pl.ds(rowbase, nrows)])

        def edge_fn(r):
            vs = gs[r, :]
            vd = gd[r, :]
            e = vs + _vgather(vd, rot8)
            e = jnp.where(e > 0.0, e, e * 0.2)
            ex = jnp.exp(e)
            exb[r, :] = ex
            for j in range(4):
                rep = _vgather(ex, hsel[j])
                gh[r, pl.ds(j * 16, 16)] = gh[r, pl.ds(j * 16, 16)] * rep

        def chunk_fn(b, j, _):
            base = b * EP + w * TPW + j * 128
            pltpu.sync_copy(srcg.at[pl.ds(base, 128)], sidx)
            pltpu.sync_copy(dstg.at[pl.ds(base, 128)], didx)
            cp1 = pltpu.async_copy(attab.at[sidx], gs, sem)
            cp2 = pltpu.async_copy(attab.at[didx], gd, sem)
            cp3 = pltpu.async_copy(htab.at[sidx], gh, sem)
            off = jnp.int32(b * NP)
            for t in range(8):
                draw[pl.ds(t * 16, 16)] = didx[pl.ds(t * 16, 16)] - off
            cp1.wait()
            cp2.wait()
            cp3.wait()
            plsc.parallel_loop(0, 128, 1, unroll=4)(edge_fn)
            pltpu.sync_copy(exb, den_acc.at[draw], add=True)
            pltpu.sync_copy(gh, agg_acc.at[draw], add=True)
            return _

        zero_accs()
        plsc.subcore_barrier()
        for b in range(NB):
            lax.fori_loop(0, NCHUNK, functools.partial(chunk_fn, b), None)
            plsc.subcore_barrier()
            pltpu.sync_copy(den_acc.at[pl.ds(rowbase, nrows)],
                            den_out.at[c, b, pl.ds(rowbase, nrows)])
            pltpu.sync_copy(agg_acc.at[pl.ds(rowbase, nrows)],
                            agg_out.at[c, b, pl.ds(rowbase, nrows)])
            if b < NB - 1:
                zero_accs()
            plsc.subcore_barrier()


def _edge1_call(attab_flat, htab_flat, srcg, dstg, z16, z64):
    mesh = plsc.VectorSubcoreMesh(core_axis_name="c", subcore_axis_name="s")
    fn = pl.kernel(
        _edge1_body,
        out_type=[
            jax.ShapeDtypeStruct((2, NB, NP, 2 * HD), jnp.float32),
            jax.ShapeDtypeStruct((2, NB, NP, HD * CH), jnp.float32),
        ],
        mesh=mesh,
        compiler_params=pltpu.CompilerParams(use_tc_tiling_on_sc=False),
        scratch_types=[
            pltpu.VMEM((128,), jnp.int32),
            pltpu.VMEM((128,), jnp.int32),
            pltpu.VMEM((128,), jnp.int32),
            pltpu.VMEM((128, 2 * HD), jnp.float32),
            pltpu.VMEM((128, 2 * HD), jnp.float32),
            pltpu.VMEM((128, HD * CH), jnp.float32),
            pltpu.VMEM((128, 2 * HD), jnp.float32),
            pltpu.SemaphoreType.DMA,
            pltpu.VMEM_SHARED((NP, 2 * HD), jnp.float32),
            pltpu.VMEM_SHARED((NP, HD * CH), jnp.float32),
        ],
    )
    return fn(attab_flat, htab_flat, srcg, dstg, z16, z64)


def _mid_body(d0_ref, d1_ref, a0_ref, a1_ref, attab_ref, b1_ref,
              g2_ref, b2_ref, w2_ref, a2s_ref, a2d_ref,
              tab2_ref, attntab_ref, stats_ref):
    ph = pl.program_id(1)
    blk = pl.program_id(2)
    den = d0_ref[0, 0] + d1_ref[0, 0]
    dsum = den[:, 0:HD]
    num = a0_ref[0, 0] + a1_ref[0, 0]
    r_i = lax.broadcasted_iota(jnp.int32, (HD, HD * CH), 0)
    c_i = lax.broadcasted_iota(jnp.int32, (HD, HD * CH), 1)
    rep = jnp.where((c_i // CH) == r_i, 1.0, 0.0)
    drep = jnp.dot(dsum, rep, preferred_element_type=jnp.float32)
    agg = num / (drep + 1e-16)
    x1 = jnp.maximum(agg + b1_ref[0, 0], 0.0)
    grow = blk * MBR + lax.broadcasted_iota(jnp.int32, (MBR, 1), 0)
    live = grow < N

    @pl.when((ph == 0) & (blk == 0))
    def _init():
        stats_ref[...] = jnp.zeros((2, HD * CH), jnp.float32)

    @pl.when(ph == 0)
    def _accum():
        x1m = jnp.where(live, x1, 0.0)
        stats_ref[0, :] += jnp.sum(x1m, axis=0)
        stats_ref[1, :] += jnp.sum(x1m * x1m, axis=0)

    m2 = stats_ref[0, :] / N
    v2 = stats_ref[1, :] / N - m2 * m2
    xn2 = (x1 - m2) / jnp.sqrt(v2 + 1e-5) * g2_ref[0, 0] + b2_ref[0, 0]
    h2 = jnp.dot(xn2, w2_ref[0], preferred_element_type=jnp.float32)
    pi = lax.broadcasted_iota(jnp.int32, (8, 16), 0)
    pj = lax.broadcasted_iota(jnp.int32, (8, 16), 1)
    P = (a2s_ref[0, 0].reshape(8, 1) * (pj == 0)
         + a2d_ref[0, 0].reshape(8, 1) * (pj == 1)
         + jnp.where(pj == pi + 2, 1.0, 0.0))
    tab2 = jnp.dot(h2, P, preferred_element_type=jnp.float32)
    di = lax.broadcasted_iota(jnp.int32, (MBR, 16), 1)
    dummy = jnp.where(di < 2, NEG, 0.0)
    tab2_ref[0] = jnp.where(live, tab2, dummy)
    attntab_ref[0] = jnp.concatenate(
        [attab_ref[0][:, HD:2 * HD], dsum], axis=1)


def _mid_call(den_out, agg_out, attab_all, b1_all, g2_all, b2_all,
              w2p_all, a2s_all, a2d_all):
    nbk = NP // MBR
    return pl.pallas_call(
        _mid_body,
        grid=(NB, 2, nbk),
        in_specs=[
            pl.BlockSpec((1, 1, MBR, 16), lambda b, p, k: (0, b, k, 0)),
            pl.BlockSpec((1, 1, MBR, 16), lambda b, p, k: (1, b, k, 0)),
            pl.BlockSpec((1, 1, MBR, 64), lambda b, p, k: (0, b, k, 0)),
            pl.BlockSpec((1, 1, MBR, 64), lambda b, p, k: (1, b, k, 0)),
            pl.BlockSpec((1, MBR, 16), lambda b, p, k: (b, k, 0)),
            pl.BlockSpec((1, 1, 64), lambda b, p, k: (b, 0, 0)),
            pl.BlockSpec((1, 1, 64), lambda b, p, k: (b, 0, 0)),
            pl.BlockSpec((1, 1, 64), lambda b, p, k: (b, 0, 0)),
            pl.BlockSpec((1, 64, 8), lambda b, p, k: (b, 0, 0)),
            pl.BlockSpec((1, 1, 8), lambda b, p, k: (b, 0, 0)),
            pl.BlockSpec((1, 1, 8), lambda b, p, k: (b, 0, 0)),
        ],
        out_specs=[
            pl.BlockSpec((1, MBR, 16), lambda b, p, k: (b, k, 0)),
            pl.BlockSpec((1, MBR, 16), lambda b, p, k: (b, k, 0)),
        ],
        out_shape=[
            jax.ShapeDtypeStruct((NB, NP, 16), jnp.float32),
            jax.ShapeDtypeStruct((NB, NP, 16), jnp.float32),
        ],
        scratch_shapes=[pltpu.VMEM((2, HD * CH), jnp.float32)],
        compiler_params=pltpu.CompilerParams(
            dimension_semantics=("arbitrary", "arbitrary", "arbitrary")),
    )(den_out, den_out, agg_out, agg_out, attab_all, b1_all,
      g2_all, b2_all, w2p_all, a2s_all, a2d_all)


def _edge2_body(tab2, attab, attntab, srcg, dstg, z16,
                acc2_out, attn_out,
                sidx, didx, gs, gd, vals, sem, acc2):
    c = lax.axis_index("c")
    s = lax.axis_index("s")
    w = s * 2 + c
    arows = NB * NP // 16
    lane = lax.iota(jnp.int32, 16)
    sel1 = lane * 0 + 1
    sel0 = lane * 0
    hsel2 = jnp.where((lane >= 8) & (lane < 12), lane - 6, 6)
    one0 = jnp.where(lane == 0, 1.0, 0.0)
    denh = 8 + (lane & 7)

    if True:
        pltpu.sync_copy(z16.at[pl.ds(0, arows)],
                        acc2.at[pl.ds(s * arows, arows)])
        plsc.subcore_barrier()

        def edge_fn(r):
            vs = gs[r, :]
            vd = gd[r, :]
            t = vs + _vgather(vd, sel1)
            e2 = _vgather(t, sel0)
            e2 = jnp.where(e2 > 0.0, e2, e2 * 0.2)
            ex2 = jnp.exp(e2)
            vals[r, :] = ex2 * (_vgather(vs, hsel2) + one0)

        def chunk_fn(b, j, _):
            base = b * EP + w * TPW + j * 128
            pltpu.sync_copy(srcg.at[pl.ds(base, 128)], sidx)
            pltpu.sync_copy(dstg.at[pl.ds(base, 128)], didx)
            cp1 = pltpu.async_copy(tab2.at[sidx], gs, sem)
            cp2 = pltpu.async_copy(tab2.at[didx], gd, sem)
            cp1.wait()
            cp2.wait()
            plsc.parallel_loop(0, 128, 1, unroll=4)(edge_fn)
            pltpu.sync_copy(vals, acc2.at[didx], add=True)
            return _

        for b in range(NB):
            lax.fori_loop(0, NCHUNK, functools.partial(chunk_fn, b), None)
        plsc.subcore_barrier()
        pltpu.sync_copy(acc2.at[pl.ds(s * arows, arows)],
                        acc2_out.at[c, pl.ds(s * arows, arows)])

        def attn_edge(r):
            vs = gs[r, :]
            vd = gd[r, :]
            e = vs + vd
            e = jnp.where(e > 0.0, e, e * 0.2)
            ex = jnp.exp(e)
            denv = _vgather(vd, denh)
            vals[r, :] = ex / (denv + 1e-16)

        def attn_chunk(j, _):
            base = w * TPW + j * 128
            pltpu.sync_copy(srcg.at[pl.ds(base, 128)], sidx)
            pltpu.sync_copy(dstg.at[pl.ds(base, 128)], didx)
            cp1 = pltpu.async_copy(attab.at[sidx], gs, sem)
            cp2 = pltpu.async_copy(attntab.at[didx], gd, sem)
            cp1.wait()
            cp2.wait()
            plsc.parallel_loop(0, 128, 1, unroll=4)(attn_edge)
            pltpu.sync_copy(vals, attn_out.at[pl.ds(base, 128)])
            return _

        lax.fori_loop(0, NCHUNK, attn_chunk, None)


def _edge2_call(tab2_flat, attab_flat, attntab, srcg, dstg, z16):
    mesh = plsc.VectorSubcoreMesh(core_axis_name="c", subcore_axis_name="s")
    fn = pl.kernel(
        _edge2_body,
        out_type=[
            jax.ShapeDtypeStruct((2, NB * NP, 16), jnp.float32),
            jax.ShapeDtypeStruct((EP, 16), jnp.float32),
        ],
        mesh=mesh,
        compiler_params=pltpu.CompilerParams(use_tc_tiling_on_sc=False),
        scratch_types=[
            pltpu.VMEM((128,), jnp.int32),
            pltpu.VMEM((128,), jnp.int32),
            pltpu.VMEM((128, 16), jnp.float32),
            pltpu.VMEM((128, 16), jnp.float32),
            pltpu.VMEM((128, 16), jnp.float32),
            pltpu.SemaphoreType.DMA,
            pltpu.VMEM_SHARED((NB * NP, 16), jnp.float32),
        ],
    )
    return fn(tab2_flat, attab_flat, attntab, srcg, dstg, z16)


def _pool_body(a20_ref, a21_ref, bb2_ref, batch_ref, ls_ref):
    A = a20_ref[0] + a21_ref[0]
    bcast = jnp.where(lax.broadcasted_iota(jnp.int32, (8, 8), 0) == 0, 1.0, 0.0)
    d = jnp.dot(A[:, 0:8], bcast, preferred_element_type=jnp.float32)
    x2 = A[:, 8:16] / (d + 1e-16) + bb2_ref[0, 0]
    x2 = jnp.where(x2 > 0.0, x2, jnp.exp(jnp.minimum(x2, 0.0)) - 1.0)
    gsel = lax.broadcasted_iota(jnp.int32, (1, 64), 1)
    oh = (batch_ref[0, 0].reshape(NP, 1) == gsel).astype(jnp.float32)
    ssum = lax.dot_general(oh, x2, (((0,), (0,)), ((), ())),
                           preferred_element_type=jnp.float32)
    cnt = jnp.sum(oh, axis=0).reshape(64, 1)
    pooled = ssum / jnp.maximum(cnt, 1.0)
    lane8 = lax.broadcasted_iota(jnp.int32, (64, 8), 1)
    mask = lane8 < 4
    mx = jnp.max(jnp.where(mask, pooled, -jnp.inf), axis=1, keepdims=True)
    se = jnp.sum(jnp.where(mask, jnp.exp(pooled - mx), 0.0),
                 axis=1, keepdims=True)
    ls_ref[0] = pooled - mx - jnp.log(se)


def _fuse_body(ls_ref, wf_ref, bf_ref, out_ref):
    c32 = jnp.concatenate([ls_ref[b] for b in range(NB)], axis=1)
    si = lax.broadcasted_iota(jnp.int32, (32, 16), 0)
    sj = lax.broadcasted_iota(jnp.int32, (32, 16), 1)
    S = jnp.where((sj == (si // 8) * 4 + (si % 8)) & (si % 8 < 4), 1.0, 0.0)
    xc = jnp.dot(c32, S, preferred_element_type=jnp.float32)
    res = jnp.dot(xc, wf_ref[...], preferred_element_type=jnp.float32) \
        + bf_ref[...]
    out_ref[...] = jnp.maximum(res[:, 0:4], 0.0)


def _final_call(a2_out, bb2_all, batch_all, wfp, bfp):
    ls = pl.pallas_call(
        _pool_body,
        grid=(NB,),
        in_specs=[
            pl.BlockSpec((1, NP, 16), lambda b: (b, 0, 0)),
            pl.BlockSpec((1, NP, 16), lambda b: (b, 0, 0)),
            pl.BlockSpec((1, 1, 8), lambda b: (b, 0, 0)),
            pl.BlockSpec((1, 1, NP), lambda b: (b, 0, 0)),
        ],
        out_specs=pl.BlockSpec((1, 64, 8), lambda b: (b, 0, 0)),
        out_shape=jax.ShapeDtypeStruct((NB, 64, 8), jnp.float32),
    )(a2_out[0].reshape(NB, NP, 16), a2_out[1].reshape(NB, NP, 16),
      bb2_all, batch_all)
    return pl.pallas_call(
        _fuse_body,
        out_shape=jax.ShapeDtypeStruct((64, 4), jnp.float32),
    )(ls, wfp, bfp)


def kernel(x_alpha, x_beta, x_theta, x_gamma, params,
           edge_index_alpha, edge_index_beta, edge_index_theta,
           edge_index_gamma, batch_alpha, batch_beta, batch_theta,
           batch_gamma):
    bands = ['alpha', 'beta', 'theta', 'gamma']
    xs = [x_alpha, x_beta, x_theta, x_gamma]
    eis = [edge_index_alpha, edge_index_beta, edge_index_theta,
           edge_index_gamma]
    bts = [batch_alpha, batch_beta, batch_theta, batch_gamma]

    x_all = jnp.stack(xs)
    g_all = jnp.stack([params[b]['bn1_g'] for b in bands]).reshape(NB, 1, F)
    b_all = jnp.stack([params[b]['bn1_b'] for b in bands]).reshape(NB, 1, F)
    w_all = jnp.stack([params[b]['conv1']['W'] for b in bands])
    asf_all = jnp.stack([params[b]['conv1']['as'].reshape(1, -1) for b in bands])
    adf_all = jnp.stack([params[b]['conv1']['ad'].reshape(1, -1) for b in bands])
    b1_all = jnp.stack([params[b]['conv1']['b'] for b in bands]).reshape(NB, 1, 64)
    g2_all = jnp.stack([params[b]['bn2_g'] for b in bands]).reshape(NB, 1, 64)
    b2_all = jnp.stack([params[b]['bn2_b'] for b in bands]).reshape(NB, 1, 64)
    w2p_all = jnp.stack([
        jnp.pad(params[b]['conv2']['W'], ((0, 0), (0, 4))) for b in bands])
    a2s_all = jnp.stack([
        jnp.pad(params[b]['conv2']['as'].reshape(-1), (0, 4)) for b in bands]).reshape(NB, 1, 8)
    a2d_all = jnp.stack([
        jnp.pad(params[b]['conv2']['ad'].reshape(-1), (0, 4)) for b in bands]).reshape(NB, 1, 8)
    bb2_all = jnp.stack([
        jnp.pad(params[b]['conv2']['b'], (0, 4)) for b in bands]).reshape(NB, 1, 8)
    wfp = jnp.pad(params['Wf'], ((0, 0), (0, 4)))
    bfp = jnp.pad(params['bf'], (0, 4))

    loop = jnp.arange(N, dtype=jnp.int32)
    padv = jnp.full((EP - E - N,), N, dtype=jnp.int32)
    srcg = jnp.concatenate([
        jnp.concatenate([ei[0], loop, padv]) + b * NP
        for b, ei in enumerate(eis)])
    dstg = jnp.concatenate([
        jnp.concatenate([ei[1], loop, padv]) + b * NP
        for b, ei in enumerate(eis)])
    batch_all = jnp.stack([
        jnp.concatenate([bt, jnp.full((NP - N,), 64, jnp.int32)])
        for bt in bts]).reshape(NB, 1, NP)

    z16 = jnp.zeros((NP, 16), jnp.float32)
    z64 = jnp.zeros((NP, 64), jnp.float32)

    attab_all, htab_all = _prep_call(x_all, g_all, b_all, w_all,
                                     asf_all, adf_all)
    attab_flat = attab_all.reshape(NB * NP, 16)
    htab_flat = htab_all.reshape(NB * NP, 64)

    den_out, agg_out = _edge1_call(attab_flat, htab_flat, srcg, dstg,
                                   z16, z64)

    tab2_all, attntab_all = _mid_call(den_out, agg_out, attab_all, b1_all,
                                      g2_all, b2_all, w2p_all,
                                      a2s_all, a2d_all)

    acc2_out, attn_wide = _edge2_call(tab2_all.reshape(NB * NP, 16),
                                      attab_flat, attntab_all[0],
                                      srcg, dstg, z16)

    out = _final_call(acc2_out, bb2_all, batch_all, wfp, bfp)
    attn = attn_wide[:E + N, 0:8]
    return out, attn


# speedup vs baseline: 78.6934x; 1.0464x over previous
import functools

import jax
import jax.numpy as jnp
from jax import lax
from jax.experimental import pallas as pl
from jax.experimental.pallas import tpu as pltpu
from jax.experimental.pallas import tpu_sc as plsc

N = 10000
NP = 10240
E = 320000
EP = 335872
F = 128
HD = 8
CH = 8
NB = 4
TPW = EP // 32
NCHUNK = TPW // 128
NEG = -1e9
MBR = 2048


def _prep_body(x_ref, g_ref, b_ref, w_ref, asf_ref, adf_ref, attab_ref, htab_ref):
    x = x_ref[0]
    m = jnp.mean(x, axis=0)
    v = jnp.mean((x - m) ** 2, axis=0)
    xn = (x - m) / jnp.sqrt(v + 1e-5) * g_ref[0, 0] + b_ref[0, 0]
    h = jnp.dot(xn, w_ref[0], preferred_element_type=jnp.float32)
    r_i = lax.broadcasted_iota(jnp.int32, (HD * CH, HD), 0)
    c_i = lax.broadcasted_iota(jnp.int32, (HD * CH, HD), 1)
    sel = (r_i // CH) == c_i
    abd_s = jnp.where(sel, asf_ref[0, 0].reshape(HD * CH, 1), 0.0)
    abd_d = jnp.where(sel, adf_ref[0, 0].reshape(HD * CH, 1), 0.0)
    as_ = jnp.dot(h, abd_s, preferred_element_type=jnp.float32)
    ad_ = jnp.dot(h, abd_d, preferred_element_type=jnp.float32)
    attab_ref[0, :N, :] = jnp.concatenate([as_, ad_], axis=1)
    attab_ref[0, N:, :] = jnp.full((NP - N, 2 * HD), NEG, jnp.float32)
    htab_ref[0, :N, :] = h
    htab_ref[0, N:, :] = jnp.zeros((NP - N, HD * CH), jnp.float32)


def _prep_call(x_all, g_all, b_all, w_all, asf_all, adf_all):
    return pl.pallas_call(
        _prep_body,
        grid=(NB,),
        in_specs=[
            pl.BlockSpec((1, N, F), lambda b: (b, 0, 0)),
            pl.BlockSpec((1, 1, F), lambda b: (b, 0, 0)),
            pl.BlockSpec((1, 1, F), lambda b: (b, 0, 0)),
            pl.BlockSpec((1, F, HD * CH), lambda b: (b, 0, 0)),
            pl.BlockSpec((1, 1, HD * CH), lambda b: (b, 0, 0)),
            pl.BlockSpec((1, 1, HD * CH), lambda b: (b, 0, 0)),
        ],
        out_specs=[
            pl.BlockSpec((1, NP, 2 * HD), lambda b: (b, 0, 0)),
            pl.BlockSpec((1, NP, HD * CH), lambda b: (b, 0, 0)),
        ],
        out_shape=[
            jax.ShapeDtypeStruct((NB, NP, 2 * HD), jnp.float32),
            jax.ShapeDtypeStruct((NB, NP, HD * CH), jnp.float32),
        ],
    )(x_all, g_all, b_all, w_all, asf_all, adf_all)


def _vgather(v, idx):
    return lax.gather(
        v, idx[:, None],
        lax.GatherDimensionNumbers(
            offset_dims=(), collapsed_slice_dims=(0,), start_index_map=(0,)),
        (1,), mode=lax.GatherScatterMode.PROMISE_IN_BOUNDS)


_ROT8 = tuple((i + 8) % 16 for i in range(16))


def _edge1_body(attab, htab, srcg, dstg, z16, z64,
                den_out, agg_out,
                sidx0, didx0, draw0, gs0, gd0, gh0, exb0,
                sidx1, didx1, draw1, gs1, gd1, gh1, exb1,
                semi0, semg0, semi1, semg1,
                den_acc, agg_acc):
    c = lax.axis_index("c")
    s = lax.axis_index("s")
    w = s * 2 + c
    rowbase = s * (NP // 16)
    nrows = NP // 16
    lane = lax.iota(jnp.int32, 16)
    rot8 = (lane + 8) & 15
    hsel = [2 * j + (lane >> 3) for j in range(4)]
    bufs = [(sidx0, didx0, draw0, gs0, gd0, gh0, exb0, semi0, semg0),
            (sidx1, didx1, draw1, gs1, gd1, gh1, exb1, semi1, semg1)]

    def zero_accs():
        pltpu.sync_copy(z16.at[pl.ds(rowbase, nrows)],
                        den_acc.at[pl.ds(rowbase, nrows)])
        pltpu.sync_copy(z64.at[pl.ds(rowbase, nrows)],
                        agg_acc.at[pl.ds(rowbase, nrows)])

    def issue_idx(b, cidx, par):
        si, di = bufs[par][0], bufs[par][1]
        semi = bufs[par][7]
        base = b * EP + w * TPW + cidx * 128
        pltpu.async_copy(srcg.at[pl.ds(base, 128)], si, semi)
        pltpu.async_copy(dstg.at[pl.ds(base, 128)], di, semi)

    def wait_idx(par):
        si, di = bufs[par][0], bufs[par][1]
        semi = bufs[par][7]
        pltpu.make_async_copy(srcg.at[pl.ds(0, 128)], si, semi).wait()
        pltpu.make_async_copy(dstg.at[pl.ds(0, 128)], di, semi).wait()

    def issue_gathers(par):
        si, di = bufs[par][0], bufs[par][1]
        gs, gd, gh = bufs[par][3], bufs[par][4], bufs[par][5]
        semg = bufs[par][8]
        pltpu.async_copy(attab.at[si], gs, semg)
        pltpu.async_copy(attab.at[di], gd, semg)
        pltpu.async_copy(htab.at[si], gh, semg)

    def wait_gathers(par):
        si = bufs[par][0]
        gs, gd, gh = bufs[par][3], bufs[par][4], bufs[par][5]
        semg = bufs[par][8]
        pltpu.make_async_copy(attab.at[si], gs, semg).wait()
        pltpu.make_async_copy(attab.at[si], gd, semg).wait()
        pltpu.make_async_copy(htab.at[si], gh, semg).wait()

    def compute_draw(b, par):
        di, draw = bufs[par][1], bufs[par][2]
        off = jnp.int32(b * NP)
        for tt in range(8):
            draw[pl.ds(tt * 16, 16)] = di[pl.ds(tt * 16, 16)] - off

    def compute_scatter(b, par):
        draw = bufs[par][2]
        gs, gd, gh, exb = bufs[par][3], bufs[par][4], bufs[par][5], bufs[par][6]

        def edge_fn(r):
            vs = gs[r, :]
            vd = gd[r, :]
            e = vs + _vgather(vd, rot8)
            e = jnp.maximum(e, e * 0.2)
            ex = jnp.exp(e)
            exb[r, :] = ex
            for j in range(4):
                rep = _vgather(ex, hsel[j])
                gh[r, pl.ds(j * 16, 16)] = gh[r, pl.ds(j * 16, 16)] * rep

        plsc.parallel_loop(0, 128, 1, unroll=4)(edge_fn)
        pltpu.sync_copy(exb, den_acc.at[draw], add=True)
        pltpu.sync_copy(gh, agg_acc.at[draw], add=True)

    zero_accs()
    plsc.subcore_barrier()
    for b in range(NB):
        issue_idx(b, 0, 0)
        wait_idx(0)
        issue_gathers(0)
        issue_idx(b, 1, 1)

        def pair_fn(t, _):
            c0 = 2 * t
            wait_idx(1)
            issue_gathers(1)
            wait_gathers(0)
            compute_draw(b, 0)

            @pl.when(c0 + 2 < NCHUNK)
            def _pre0():
                issue_idx(b, c0 + 2, 0)
            compute_scatter(b, 0)

            @pl.when(c0 + 2 < NCHUNK)
            def _g0():
                wait_idx(0)
                issue_gathers(0)
            wait_gathers(1)
            compute_draw(b, 1)

            @pl.when(c0 + 3 < NCHUNK)
            def _pre1():
                issue_idx(b, c0 + 3, 1)
            compute_scatter(b, 1)
            return _

        lax.fori_loop(0, NCHUNK // 2, pair_fn, None)
        plsc.subcore_barrier()
        pltpu.sync_copy(den_acc.at[pl.ds(rowbase, nrows)],
                        den_out.at[c, b, pl.ds(rowbase, nrows)])
        pltpu.sync_copy(agg_acc.at[pl.ds(rowbase, nrows)],
                        agg_out.at[c, b, pl.ds(rowbase, nrows)])
        if b < NB - 1:
            zero_accs()
        plsc.subcore_barrier()


def _edge1_call(attab_flat, htab_flat, srcg, dstg, z16, z64):
    mesh = plsc.VectorSubcoreMesh(core_axis_name="c", subcore_axis_name="s")
    fn = pl.kernel(
        _edge1_body,
        out_type=[
            jax.ShapeDtypeStruct((2, NB, NP, 2 * HD), jnp.float32),
            jax.ShapeDtypeStruct((2, NB, NP, HD * CH), jnp.float32),
        ],
        mesh=mesh,
        compiler_params=pltpu.CompilerParams(use_tc_tiling_on_sc=False),
        scratch_types=(
            [pltpu.VMEM((128,), jnp.int32),
             pltpu.VMEM((128,), jnp.int32),
             pltpu.VMEM((128,), jnp.int32),
             pltpu.VMEM((128, 2 * HD), jnp.float32),
             pltpu.VMEM((128, 2 * HD), jnp.float32),
             pltpu.VMEM((128, HD * CH), jnp.float32),
             pltpu.VMEM((128, 2 * HD), jnp.float32)] * 2
            + [pltpu.SemaphoreType.DMA] * 4
            + [pltpu.VMEM_SHARED((NP, 2 * HD), jnp.float32),
               pltpu.VMEM_SHARED((NP, HD * CH), jnp.float32)]
        ),
    )
    return fn(attab_flat, htab_flat, srcg, dstg, z16, z64)


def _mid_body(d0_ref, d1_ref, a0_ref, a1_ref, attab_ref, b1_ref,
              g2_ref, b2_ref, w2_ref, a2s_ref, a2d_ref,
              tab2_ref, attntab_ref, stats_ref):
    ph = pl.program_id(1)
    blk = pl.program_id(2)
    den = d0_ref[0, 0] + d1_ref[0, 0]
    dsum = den[:, 0:HD]
    num = a0_ref[0, 0] + a1_ref[0, 0]
    r_i = lax.broadcasted_iota(jnp.int32, (HD, HD * CH), 0)
    c_i = lax.broadcasted_iota(jnp.int32, (HD, HD * CH), 1)
    rep = jnp.where((c_i // CH) == r_i, 1.0, 0.0)
    drep = jnp.dot(dsum, rep, preferred_element_type=jnp.float32)
    agg = num / (drep + 1e-16)
    x1 = jnp.maximum(agg + b1_ref[0, 0], 0.0)
    grow = blk * MBR + lax.broadcasted_iota(jnp.int32, (MBR, 1), 0)
    live = grow < N

    @pl.when((ph == 0) & (blk == 0))
    def _init():
        stats_ref[...] = jnp.zeros((2, HD * CH), jnp.float32)

    @pl.when(ph == 0)
    def _accum():
        x1m = jnp.where(live, x1, 0.0)
        stats_ref[0, :] += jnp.sum(x1m, axis=0)
        stats_ref[1, :] += jnp.sum(x1m * x1m, axis=0)

    m2 = stats_ref[0, :] / N
    v2 = stats_ref[1, :] / N - m2 * m2
    xn2 = (x1 - m2) / jnp.sqrt(v2 + 1e-5) * g2_ref[0, 0] + b2_ref[0, 0]
    h2 = jnp.dot(xn2, w2_ref[0], preferred_element_type=jnp.float32)
    pi = lax.broadcasted_iota(jnp.int32, (8, 16), 0)
    pj = lax.broadcasted_iota(jnp.int32, (8, 16), 1)
    P = (a2s_ref[0, 0].reshape(8, 1) * (pj == 0)
         + a2d_ref[0, 0].reshape(8, 1) * (pj == 1)
         + jnp.where(pj == pi + 2, 1.0, 0.0))
    tab2 = jnp.dot(h2, P, preferred_element_type=jnp.float32)
    di = lax.broadcasted_iota(jnp.int32, (MBR, 16), 1)
    dummy = jnp.where(di < 2, NEG, 0.0)
    tab2_ref[0] = jnp.where(live, tab2, dummy)
    attntab_ref[0] = jnp.concatenate(
        [attab_ref[0][:, HD:2 * HD], dsum], axis=1)


def _mid_call(den_out, agg_out, attab_all, b1_all, g2_all, b2_all,
              w2p_all, a2s_all, a2d_all):
    nbk = NP // MBR
    return pl.pallas_call(
        _mid_body,
        grid=(NB, 2, nbk),
        in_specs=[
            pl.BlockSpec((1, 1, MBR, 16), lambda b, p, k: (0, b, k, 0)),
            pl.BlockSpec((1, 1, MBR, 16), lambda b, p, k: (1, b, k, 0)),
            pl.BlockSpec((1, 1, MBR, 64), lambda b, p, k: (0, b, k, 0)),
            pl.BlockSpec((1, 1, MBR, 64), lambda b, p, k: (1, b, k, 0)),
            pl.BlockSpec((1, MBR, 16), lambda b, p, k: (b, k, 0)),
            pl.BlockSpec((1, 1, 64), lambda b, p, k: (b, 0, 0)),
            pl.BlockSpec((1, 1, 64), lambda b, p, k: (b, 0, 0)),
            pl.BlockSpec((1, 1, 64), lambda b, p, k: (b, 0, 0)),
            pl.BlockSpec((1, 64, 8), lambda b, p, k: (b, 0, 0)),
            pl.BlockSpec((1, 1, 8), lambda b, p, k: (b, 0, 0)),
            pl.BlockSpec((1, 1, 8), lambda b, p, k: (b, 0, 0)),
        ],
        out_specs=[
            pl.BlockSpec((1, MBR, 16), lambda b, p, k: (b, k, 0)),
            pl.BlockSpec((1, MBR, 16), lambda b, p, k: (b, k, 0)),
        ],
        out_shape=[
            jax.ShapeDtypeStruct((NB, NP, 16), jnp.float32),
            jax.ShapeDtypeStruct((NB, NP, 16), jnp.float32),
        ],
        scratch_shapes=[pltpu.VMEM((2, HD * CH), jnp.float32)],
        compiler_params=pltpu.CompilerParams(
            dimension_semantics=("arbitrary", "arbitrary", "arbitrary")),
    )(den_out, den_out, agg_out, agg_out, attab_all, b1_all,
      g2_all, b2_all, w2p_all, a2s_all, a2d_all)


def _edge2_body(tab2, attab, attntab, srcg, dstg, z16,
                acc2_out, attn_out,
                sidx, didx, gs, gd, vals, sem, acc2):
    c = lax.axis_index("c")
    s = lax.axis_index("s")
    w = s * 2 + c
    arows = NB * NP // 16
    lane = lax.iota(jnp.int32, 16)
    sel1 = lane * 0 + 1
    sel0 = lane * 0
    hsel2 = jnp.where((lane >= 8) & (lane < 12), lane - 6, 6)
    one0 = jnp.where(lane == 0, 1.0, 0.0)
    denh = 8 + (lane & 7)

    if True:
        pltpu.sync_copy(z16.at[pl.ds(0, arows)],
                        acc2.at[pl.ds(s * arows, arows)])
        plsc.subcore_barrier()

        def edge_fn(r):
            vs = gs[r, :]
            vd = gd[r, :]
            t = vs + _vgather(vd, sel1)
            e2 = _vgather(t, sel0)
            e2 = jnp.where(e2 > 0.0, e2, e2 * 0.2)
            ex2 = jnp.exp(e2)
            vals[r, :] = ex2 * (_vgather(vs, hsel2) + one0)

        def chunk_fn(b, j, _):
            base = b * EP + w * TPW + j * 128
            pltpu.sync_copy(srcg.at[pl.ds(base, 128)], sidx)
            pltpu.sync_copy(dstg.at[pl.ds(base, 128)], didx)
            cp1 = pltpu.async_copy(tab2.at[sidx], gs, sem)
            cp2 = pltpu.async_copy(tab2.at[didx], gd, sem)
            cp1.wait()
            cp2.wait()
            plsc.parallel_loop(0, 128, 1, unroll=4)(edge_fn)
            pltpu.sync_copy(vals, acc2.at[didx], add=True)
            return _

        for b in range(NB):
            lax.fori_loop(0, NCHUNK, functools.partial(chunk_fn, b), None)
        plsc.subcore_barrier()
        pltpu.sync_copy(acc2.at[pl.ds(s * arows, arows)],
                        acc2_out.at[c, pl.ds(s * arows, arows)])

        def attn_edge(r):
            vs = gs[r, :]
            vd = gd[r, :]
            e = vs + vd
            e = jnp.where(e > 0.0, e, e * 0.2)
            ex = jnp.exp(e)
            denv = _vgather(vd, denh)
            vals[r, :] = ex / (denv + 1e-16)

        def attn_chunk(j, _):
            base = w * TPW + j * 128
            pltpu.sync_copy(srcg.at[pl.ds(base, 128)], sidx)
            pltpu.sync_copy(dstg.at[pl.ds(base, 128)], didx)
            cp1 = pltpu.async_copy(attab.at[sidx], gs, sem)
            cp2 = pltpu.async_copy(attntab.at[didx], gd, sem)
            cp1.wait()
            cp2.wait()
            plsc.parallel_loop(0, 128, 1, unroll=4)(attn_edge)
            pltpu.sync_copy(vals, attn_out.at[pl.ds(base, 128)])
            return _

        lax.fori_loop(0, NCHUNK, attn_chunk, None)


def _edge2_call(tab2_flat, attab_flat, attntab, srcg, dstg, z16):
    mesh = plsc.VectorSubcoreMesh(core_axis_name="c", subcore_axis_name="s")
    fn = pl.kernel(
        _edge2_body,
        out_type=[
            jax.ShapeDtypeStruct((2, NB * NP, 16), jnp.float32),
            jax.ShapeDtypeStruct((EP, 16), jnp.float32),
        ],
        mesh=mesh,
        compiler_params=pltpu.CompilerParams(use_tc_tiling_on_sc=False),
        scratch_types=[
            pltpu.VMEM((128,), jnp.int32),
            pltpu.VMEM((128,), jnp.int32),
            pltpu.VMEM((128, 16), jnp.float32),
            pltpu.VMEM((128, 16), jnp.float32),
            pltpu.VMEM((128, 16), jnp.float32),
            pltpu.SemaphoreType.DMA,
            pltpu.VMEM_SHARED((NB * NP, 16), jnp.float32),
        ],
    )
    return fn(tab2_flat, attab_flat, attntab, srcg, dstg, z16)


def _pool_body(a20_ref, a21_ref, bb2_ref, batch_ref, ls_ref):
    A = a20_ref[0] + a21_ref[0]
    bcast = jnp.where(lax.broadcasted_iota(jnp.int32, (8, 8), 0) == 0, 1.0, 0.0)
    d = jnp.dot(A[:, 0:8], bcast, preferred_element_type=jnp.float32)
    x2 = A[:, 8:16] / (d + 1e-16) + bb2_ref[0, 0]
    x2 = jnp.where(x2 > 0.0, x2, jnp.exp(jnp.minimum(x2, 0.0)) - 1.0)
    gsel = lax.broadcasted_iota(jnp.int32, (1, 64), 1)
    oh = (batch_ref[0, 0].reshape(NP, 1) == gsel).astype(jnp.float32)
    ssum = lax.dot_general(oh, x2, (((0,), (0,)), ((), ())),
                           preferred_element_type=jnp.float32)
    cnt = jnp.sum(oh, axis=0).reshape(64, 1)
    pooled = ssum / jnp.maximum(cnt, 1.0)
    lane8 = lax.broadcasted_iota(jnp.int32, (64, 8), 1)
    mask = lane8 < 4
    mx = jnp.max(jnp.where(mask, pooled, -jnp.inf), axis=1, keepdims=True)
    se = jnp.sum(jnp.where(mask, jnp.exp(pooled - mx), 0.0),
                 axis=1, keepdims=True)
    ls_ref[0] = pooled - mx - jnp.log(se)


def _fuse_body(ls_ref, wf_ref, bf_ref, out_ref):
    c32 = jnp.concatenate([ls_ref[b] for b in range(NB)], axis=1)
    si = lax.broadcasted_iota(jnp.int32, (32, 16), 0)
    sj = lax.broadcasted_iota(jnp.int32, (32, 16), 1)
    S = jnp.where((sj == (si // 8) * 4 + (si % 8)) & (si % 8 < 4), 1.0, 0.0)
    xc = jnp.dot(c32, S, preferred_element_type=jnp.float32)
    res = jnp.dot(xc, wf_ref[...], preferred_element_type=jnp.float32) \
        + bf_ref[...]
    out_ref[...] = jnp.maximum(res[:, 0:4], 0.0)


def _final_call(a2_out, bb2_all, batch_all, wfp, bfp):
    ls = pl.pallas_call(
        _pool_body,
        grid=(NB,),
        in_specs=[
            pl.BlockSpec((1, NP, 16), lambda b: (b, 0, 0)),
            pl.BlockSpec((1, NP, 16), lambda b: (b, 0, 0)),
            pl.BlockSpec((1, 1, 8), lambda b: (b, 0, 0)),
            pl.BlockSpec((1, 1, NP), lambda b: (b, 0, 0)),
        ],
        out_specs=pl.BlockSpec((1, 64, 8), lambda b: (b, 0, 0)),
        out_shape=jax.ShapeDtypeStruct((NB, 64, 8), jnp.float32),
    )(a2_out[0].reshape(NB, NP, 16), a2_out[1].reshape(NB, NP, 16),
      bb2_all, batch_all)
    return pl.pallas_call(
        _fuse_body,
        out_shape=jax.ShapeDtypeStruct((64, 4), jnp.float32),
    )(ls, wfp, bfp)


def kernel(x_alpha, x_beta, x_theta, x_gamma, params,
           edge_index_alpha, edge_index_beta, edge_index_theta,
           edge_index_gamma, batch_alpha, batch_beta, batch_theta,
           batch_gamma):
    bands = ['alpha', 'beta', 'theta', 'gamma']
    xs = [x_alpha, x_beta, x_theta, x_gamma]
    eis = [edge_index_alpha, edge_index_beta, edge_index_theta,
           edge_index_gamma]
    bts = [batch_alpha, batch_beta, batch_theta, batch_gamma]

    x_all = jnp.stack(xs)
    g_all = jnp.stack([params[b]['bn1_g'] for b in bands]).reshape(NB, 1, F)
    b_all = jnp.stack([params[b]['bn1_b'] for b in bands]).reshape(NB, 1, F)
    w_all = jnp.stack([params[b]['conv1']['W'] for b in bands])
    asf_all = jnp.stack([params[b]['conv1']['as'].reshape(1, -1) for b in bands])
    adf_all = jnp.stack([params[b]['conv1']['ad'].reshape(1, -1) for b in bands])
    b1_all = jnp.stack([params[b]['conv1']['b'] for b in bands]).reshape(NB, 1, 64)
    g2_all = jnp.stack([params[b]['bn2_g'] for b in bands]).reshape(NB, 1, 64)
    b2_all = jnp.stack([params[b]['bn2_b'] for b in bands]).reshape(NB, 1, 64)
    w2p_all = jnp.stack([
        jnp.pad(params[b]['conv2']['W'], ((0, 0), (0, 4))) for b in bands])
    a2s_all = jnp.stack([
        jnp.pad(params[b]['conv2']['as'].reshape(-1), (0, 4)) for b in bands]).reshape(NB, 1, 8)
    a2d_all = jnp.stack([
        jnp.pad(params[b]['conv2']['ad'].reshape(-1), (0, 4)) for b in bands]).reshape(NB, 1, 8)
    bb2_all = jnp.stack([
        jnp.pad(params[b]['conv2']['b'], (0, 4)) for b in bands]).reshape(NB, 1, 8)
    wfp = jnp.pad(params['Wf'], ((0, 0), (0, 4)))
    bfp = jnp.pad(params['bf'], (0, 4))

    loop = jnp.arange(N, dtype=jnp.int32)
    padv = jnp.full((EP - E - N,), N, dtype=jnp.int32)
    srcg = jnp.concatenate([
        jnp.concatenate([ei[0], loop, padv]) + b * NP
        for b, ei in enumerate(eis)])
    dstg = jnp.concatenate([
        jnp.concatenate([ei[1], loop, padv]) + b * NP
        for b, ei in enumerate(eis)])
    batch_all = jnp.stack([
        jnp.concatenate([bt, jnp.full((NP - N,), 64, jnp.int32)])
        for bt in bts]).reshape(NB, 1, NP)

    z16 = jnp.zeros((NP, 16), jnp.float32)
    z64 = jnp.zeros((NP, 64), jnp.float32)

    attab_all, htab_all = _prep_call(x_all, g_all, b_all, w_all,
                                     asf_all, adf_all)
    attab_flat = attab_all.reshape(NB * NP, 16)
    htab_flat = htab_all.reshape(NB * NP, 64)

    den_out, agg_out = _edge1_call(attab_flat, htab_flat, srcg, dstg,
                                   z16, z64)

    tab2_all, attntab_all = _mid_call(den_out, agg_out, attab_all, b1_all,
                                      g2_all, b2_all, w2p_all,
                                      a2s_all, a2d_all)

    acc2_out, attn_wide = _edge2_call(tab2_all.reshape(NB * NP, 16),
                                      attab_flat, attntab_all[0],
                                      srcg, dstg, z16)

    out = _final_call(acc2_out, bb2_all, batch_all, wfp, bfp)
    attn = attn_wide[:E + N, 0:8]
    return out, attn


# speedup vs baseline: 100.4130x; 1.2760x over previous
import functools

import jax
import jax.numpy as jnp
from jax import lax
from jax.experimental import pallas as pl
from jax.experimental.pallas import tpu as pltpu
from jax.experimental.pallas import tpu_sc as plsc

N = 10000
NP = 10240
E = 320000
EP = 335872
F = 128
HD = 8
CH = 8
NB = 4
TPW = EP // 32
NCHUNK = TPW // 128
NEG = -1e9
MBR = 2048


def _prep_body(x_ref, g_ref, b_ref, w_ref, asf_ref, adf_ref, attab_ref, htab_ref):
    x = x_ref[0]
    m = jnp.mean(x, axis=0)
    v = jnp.mean((x - m) ** 2, axis=0)
    xn = (x - m) / jnp.sqrt(v + 1e-5) * g_ref[0, 0] + b_ref[0, 0]
    h = jnp.dot(xn, w_ref[0], preferred_element_type=jnp.float32)
    r_i = lax.broadcasted_iota(jnp.int32, (HD * CH, HD), 0)
    c_i = lax.broadcasted_iota(jnp.int32, (HD * CH, HD), 1)
    sel = (r_i // CH) == c_i
    abd_s = jnp.where(sel, asf_ref[0, 0].reshape(HD * CH, 1), 0.0)
    abd_d = jnp.where(sel, adf_ref[0, 0].reshape(HD * CH, 1), 0.0)
    as_ = jnp.dot(h, abd_s, preferred_element_type=jnp.float32)
    ad_ = jnp.dot(h, abd_d, preferred_element_type=jnp.float32)
    attab_ref[0, :N, :] = jnp.concatenate([as_, ad_], axis=1)
    attab_ref[0, N:, :] = jnp.full((NP - N, 2 * HD), NEG, jnp.float32)
    htab_ref[0, :N, :] = h
    htab_ref[0, N:, :] = jnp.zeros((NP - N, HD * CH), jnp.float32)


def _prep_call(x_all, g_all, b_all, w_all, asf_all, adf_all):
    return pl.pallas_call(
        _prep_body,
        grid=(NB,),
        in_specs=[
            pl.BlockSpec((1, N, F), lambda b: (b, 0, 0)),
            pl.BlockSpec((1, 1, F), lambda b: (b, 0, 0)),
            pl.BlockSpec((1, 1, F), lambda b: (b, 0, 0)),
            pl.BlockSpec((1, F, HD * CH), lambda b: (b, 0, 0)),
            pl.BlockSpec((1, 1, HD * CH), lambda b: (b, 0, 0)),
            pl.BlockSpec((1, 1, HD * CH), lambda b: (b, 0, 0)),
        ],
        out_specs=[
            pl.BlockSpec((1, NP, 2 * HD), lambda b: (b, 0, 0)),
            pl.BlockSpec((1, NP, HD * CH), lambda b: (b, 0, 0)),
        ],
        out_shape=[
            jax.ShapeDtypeStruct((NB, NP, 2 * HD), jnp.float32),
            jax.ShapeDtypeStruct((NB, NP, HD * CH), jnp.float32),
        ],
    )(x_all, g_all, b_all, w_all, asf_all, adf_all)


def _vgather(v, idx):
    return lax.gather(
        v, idx[:, None],
        lax.GatherDimensionNumbers(
            offset_dims=(), collapsed_slice_dims=(0,), start_index_map=(0,)),
        (1,), mode=lax.GatherScatterMode.PROMISE_IN_BOUNDS)


_ROT8 = tuple((i + 8) % 16 for i in range(16))


def _edge1_body(attab, htab, srcg, dstg, z16, z64,
                den_out, agg_out,
                sidx0, didx0, draw0, gs0, gd0, gh0, exb0,
                sidx1, didx1, draw1, gs1, gd1, gh1, exb1,
                semi0, semg0, semi1, semg1,
                den_acc, agg_acc):
    c = lax.axis_index("c")
    s = lax.axis_index("s")
    w = s * 2 + c
    rowbase = s * (NP // 16)
    nrows = NP // 16
    lane = lax.iota(jnp.int32, 16)
    rot8 = (lane + 8) & 15
    hsel = [2 * j + (lane >> 3) for j in range(4)]
    bufs = [(sidx0, didx0, draw0, gs0, gd0, gh0, exb0, semi0, semg0),
            (sidx1, didx1, draw1, gs1, gd1, gh1, exb1, semi1, semg1)]

    def zero_accs():
        pltpu.sync_copy(z16.at[pl.ds(rowbase, nrows)],
                        den_acc.at[pl.ds(rowbase, nrows)])
        pltpu.sync_copy(z64.at[pl.ds(rowbase, nrows)],
                        agg_acc.at[pl.ds(rowbase, nrows)])

    def issue_idx(b, cidx, par):
        si, di = bufs[par][0], bufs[par][1]
        semi = bufs[par][7]
        base = b * EP + w * TPW + cidx * 128
        pltpu.async_copy(srcg.at[pl.ds(base, 128)], si, semi)
        pltpu.async_copy(dstg.at[pl.ds(base, 128)], di, semi)

    def wait_idx(par):
        si, di = bufs[par][0], bufs[par][1]
        semi = bufs[par][7]
        pltpu.make_async_copy(srcg.at[pl.ds(0, 128)], si, semi).wait()
        pltpu.make_async_copy(dstg.at[pl.ds(0, 128)], di, semi).wait()

    def issue_gathers(par):
        si, di = bufs[par][0], bufs[par][1]
        gs, gd, gh = bufs[par][3], bufs[par][4], bufs[par][5]
        semg = bufs[par][8]
        pltpu.async_copy(attab.at[si], gs, semg)
        pltpu.async_copy(attab.at[di], gd, semg)
        pltpu.async_copy(htab.at[si], gh, semg)

    def wait_gathers(par):
        si = bufs[par][0]
        gs, gd, gh = bufs[par][3], bufs[par][4], bufs[par][5]
        semg = bufs[par][8]
        pltpu.make_async_copy(attab.at[si], gs, semg).wait()
        pltpu.make_async_copy(attab.at[si], gd, semg).wait()
        pltpu.make_async_copy(htab.at[si], gh, semg).wait()

    def compute_draw(b, par):
        di, draw = bufs[par][1], bufs[par][2]
        off = jnp.int32(b * NP)
        for tt in range(8):
            draw[pl.ds(tt * 16, 16)] = di[pl.ds(tt * 16, 16)] - off

    def compute_scatter(b, par):
        draw = bufs[par][2]
        gs, gd, gh, exb = bufs[par][3], bufs[par][4], bufs[par][5], bufs[par][6]

        def edge_fn(r):
            vs = gs[r, :]
            vd = gd[r, :]
            e = vs + _vgather(vd, rot8)
            e = jnp.maximum(e, e * 0.2)
            ex = jnp.exp(e)
            exb[r, :] = ex
            for j in range(4):
                rep = _vgather(ex, hsel[j])
                gh[r, pl.ds(j * 16, 16)] = gh[r, pl.ds(j * 16, 16)] * rep

        plsc.parallel_loop(0, 128, 1, unroll=4)(edge_fn)
        pltpu.sync_copy(exb, den_acc.at[draw], add=True)
        pltpu.sync_copy(gh, agg_acc.at[draw], add=True)

    zero_accs()
    plsc.subcore_barrier()
    for b in range(NB):
        issue_idx(b, 0, 0)
        wait_idx(0)
        issue_gathers(0)
        issue_idx(b, 1, 1)

        def pair_fn(t, _):
            c0 = 2 * t
            wait_idx(1)
            issue_gathers(1)
            wait_gathers(0)
            compute_draw(b, 0)

            @pl.when(c0 + 2 < NCHUNK)
            def _pre0():
                issue_idx(b, c0 + 2, 0)
            compute_scatter(b, 0)

            @pl.when(c0 + 2 < NCHUNK)
            def _g0():
                wait_idx(0)
                issue_gathers(0)
            wait_gathers(1)
            compute_draw(b, 1)

            @pl.when(c0 + 3 < NCHUNK)
            def _pre1():
                issue_idx(b, c0 + 3, 1)
            compute_scatter(b, 1)
            return _

        lax.fori_loop(0, NCHUNK // 2, pair_fn, None)
        plsc.subcore_barrier()
        pltpu.sync_copy(den_acc.at[pl.ds(rowbase, nrows)],
                        den_out.at[c, b, pl.ds(rowbase, nrows)])
        pltpu.sync_copy(agg_acc.at[pl.ds(rowbase, nrows)],
                        agg_out.at[c, b, pl.ds(rowbase, nrows)])
        if b < NB - 1:
            zero_accs()
        plsc.subcore_barrier()


def _edge1_call(attab_flat, htab_flat, srcg, dstg, z16, z64):
    mesh = plsc.VectorSubcoreMesh(core_axis_name="c", subcore_axis_name="s")
    fn = pl.kernel(
        _edge1_body,
        out_type=[
            jax.ShapeDtypeStruct((2, NB, NP, 2 * HD), jnp.float32),
            jax.ShapeDtypeStruct((2, NB, NP, HD * CH), jnp.float32),
        ],
        mesh=mesh,
        compiler_params=pltpu.CompilerParams(use_tc_tiling_on_sc=False),
        scratch_types=(
            [pltpu.VMEM((128,), jnp.int32),
             pltpu.VMEM((128,), jnp.int32),
             pltpu.VMEM((128,), jnp.int32),
             pltpu.VMEM((128, 2 * HD), jnp.float32),
             pltpu.VMEM((128, 2 * HD), jnp.float32),
             pltpu.VMEM((128, HD * CH), jnp.float32),
             pltpu.VMEM((128, 2 * HD), jnp.float32)] * 2
            + [pltpu.SemaphoreType.DMA] * 4
            + [pltpu.VMEM_SHARED((NP, 2 * HD), jnp.float32),
               pltpu.VMEM_SHARED((NP, HD * CH), jnp.float32)]
        ),
    )
    return fn(attab_flat, htab_flat, srcg, dstg, z16, z64)


def _mid_body(d0_ref, d1_ref, a0_ref, a1_ref, attab_ref, b1_ref,
              g2_ref, b2_ref, w2_ref, a2s_ref, a2d_ref,
              tab2_ref, attntab_ref, stats_ref):
    ph = pl.program_id(1)
    blk = pl.program_id(2)
    den = d0_ref[0, 0] + d1_ref[0, 0]
    dsum = den[:, 0:HD]
    num = a0_ref[0, 0] + a1_ref[0, 0]
    r_i = lax.broadcasted_iota(jnp.int32, (HD, HD * CH), 0)
    c_i = lax.broadcasted_iota(jnp.int32, (HD, HD * CH), 1)
    rep = jnp.where((c_i // CH) == r_i, 1.0, 0.0)
    drep = jnp.dot(dsum, rep, preferred_element_type=jnp.float32)
    agg = num / (drep + 1e-16)
    x1 = jnp.maximum(agg + b1_ref[0, 0], 0.0)
    grow = blk * MBR + lax.broadcasted_iota(jnp.int32, (MBR, 1), 0)
    live = grow < N

    @pl.when((ph == 0) & (blk == 0))
    def _init():
        stats_ref[...] = jnp.zeros((2, HD * CH), jnp.float32)

    @pl.when(ph == 0)
    def _accum_mean():
        x1m = jnp.where(live, x1, 0.0)
        stats_ref[0, :] += jnp.sum(x1m, axis=0)

    m2 = stats_ref[0, :] / N

    @pl.when(ph == 1)
    def _accum_var():
        d = jnp.where(live, x1 - m2, 0.0)
        stats_ref[1, :] += jnp.sum(d * d, axis=0)

    v2 = stats_ref[1, :] / N
    xn2 = (x1 - m2) / jnp.sqrt(v2 + 1e-5) * g2_ref[0, 0] + b2_ref[0, 0]
    h2 = jnp.dot(xn2, w2_ref[0], preferred_element_type=jnp.float32)
    pi = lax.broadcasted_iota(jnp.int32, (8, 16), 0)
    pj = lax.broadcasted_iota(jnp.int32, (8, 16), 1)
    P = (a2s_ref[0, 0].reshape(8, 1) * (pj == 0)
         + a2d_ref[0, 0].reshape(8, 1) * (pj == 1)
         + jnp.where(pj == pi + 2, 1.0, 0.0))
    tab2 = jnp.dot(h2, P, preferred_element_type=jnp.float32)
    di = lax.broadcasted_iota(jnp.int32, (MBR, 16), 1)
    dummy = jnp.where(di < 2, NEG, 0.0)
    tab2_ref[0] = jnp.where(live, tab2, dummy)
    attntab_ref[0] = jnp.concatenate(
        [attab_ref[0][:, HD:2 * HD], dsum], axis=1)


def _mid_call(den_out, agg_out, attab_all, b1_all, g2_all, b2_all,
              w2p_all, a2s_all, a2d_all):
    nbk = NP // MBR
    return pl.pallas_call(
        _mid_body,
        grid=(NB, 3, nbk),
        in_specs=[
            pl.BlockSpec((1, 1, MBR, 16), lambda b, p, k: (0, b, k, 0)),
            pl.BlockSpec((1, 1, MBR, 16), lambda b, p, k: (1, b, k, 0)),
            pl.BlockSpec((1, 1, MBR, 64), lambda b, p, k: (0, b, k, 0)),
            pl.BlockSpec((1, 1, MBR, 64), lambda b, p, k: (1, b, k, 0)),
            pl.BlockSpec((1, MBR, 16), lambda b, p, k: (b, k, 0)),
            pl.BlockSpec((1, 1, 64), lambda b, p, k: (b, 0, 0)),
            pl.BlockSpec((1, 1, 64), lambda b, p, k: (b, 0, 0)),
            pl.BlockSpec((1, 1, 64), lambda b, p, k: (b, 0, 0)),
            pl.BlockSpec((1, 64, 8), lambda b, p, k: (b, 0, 0)),
            pl.BlockSpec((1, 1, 8), lambda b, p, k: (b, 0, 0)),
            pl.BlockSpec((1, 1, 8), lambda b, p, k: (b, 0, 0)),
        ],
        out_specs=[
            pl.BlockSpec((1, MBR, 16), lambda b, p, k: (b, k, 0)),
            pl.BlockSpec((1, MBR, 16), lambda b, p, k: (b, k, 0)),
        ],
        out_shape=[
            jax.ShapeDtypeStruct((NB, NP, 16), jnp.float32),
            jax.ShapeDtypeStruct((NB, NP, 16), jnp.float32),
        ],
        scratch_shapes=[pltpu.VMEM((2, HD * CH), jnp.float32)],
        compiler_params=pltpu.CompilerParams(
            dimension_semantics=("arbitrary", "arbitrary", "arbitrary")),
    )(den_out, den_out, agg_out, agg_out, attab_all, b1_all,
      g2_all, b2_all, w2p_all, a2s_all, a2d_all)


def _edge2_body(tab2, attab, attntab, srcg, dstg, z16,
                acc2_out, attn_out,
                sidx0, didx0, dscat0, gs0, gd0, vals0,
                sidx1, didx1, dscat1, gs1, gd1, vals1,
                semi0, semg0, semi1, semg1,
                acc2):
    c = lax.axis_index("c")
    s = lax.axis_index("s")
    w = s * 2 + c
    arows = NB * NP // 16
    lane = lax.iota(jnp.int32, 16)
    sel1 = lane * 0 + 1
    sel0 = lane * 0
    hsel2 = jnp.where((lane >= 8) & (lane < 12), lane - 6, 6)
    one0 = jnp.where(lane == 0, 1.0, 0.0)
    denh = 8 + (lane & 7)
    bufs = [(sidx0, didx0, dscat0, gs0, gd0, vals0, semi0, semg0),
            (sidx1, didx1, dscat1, gs1, gd1, vals1, semi1, semg1)]

    def issue_idx(base0, cidx, par):
        si, di, semi = bufs[par][0], bufs[par][1], bufs[par][6]
        base = base0 + cidx * 128
        pltpu.async_copy(srcg.at[pl.ds(base, 128)], si, semi)
        pltpu.async_copy(dstg.at[pl.ds(base, 128)], di, semi)

    def wait_idx(par):
        si, di, semi = bufs[par][0], bufs[par][1], bufs[par][6]
        pltpu.make_async_copy(srcg.at[pl.ds(0, 128)], si, semi).wait()
        pltpu.make_async_copy(dstg.at[pl.ds(0, 128)], di, semi).wait()

    def issue_gathers(ta, tb, par):
        si, di = bufs[par][0], bufs[par][1]
        gs, gd, semg = bufs[par][3], bufs[par][4], bufs[par][7]
        pltpu.async_copy(ta.at[si], gs, semg)
        pltpu.async_copy(tb.at[di], gd, semg)

    def wait_gathers(ta, tb, par):
        si = bufs[par][0]
        gs, gd, semg = bufs[par][3], bufs[par][4], bufs[par][7]
        pltpu.make_async_copy(ta.at[si], gs, semg).wait()
        pltpu.make_async_copy(ta.at[si], gd, semg).wait()

    def snap_dst(par):
        di, dscat = bufs[par][1], bufs[par][2]
        for tt in range(8):
            dscat[pl.ds(tt * 16, 16)] = di[pl.ds(tt * 16, 16)]

    def conv2_compute(par):
        gs, gd, vals = bufs[par][3], bufs[par][4], bufs[par][5]
        dscat = bufs[par][2]

        def edge_fn(r):
            vs = gs[r, :]
            vd = gd[r, :]
            tt = vs + _vgather(vd, sel1)
            e2 = _vgather(tt, sel0)
            e2 = jnp.maximum(e2, e2 * 0.2)
            ex2 = jnp.exp(e2)
            vals[r, :] = ex2 * (_vgather(vs, hsel2) + one0)

        plsc.parallel_loop(0, 128, 1, unroll=4)(edge_fn)
        pltpu.sync_copy(vals, acc2.at[dscat], add=True)

    def attn_compute(base0, cidx, par):
        gs, gd, vals = bufs[par][3], bufs[par][4], bufs[par][5]

        def edge_fn(r):
            vs = gs[r, :]
            vd = gd[r, :]
            e = vs + vd
            e = jnp.maximum(e, e * 0.2)
            ex = jnp.exp(e)
            denv = _vgather(vd, denh)
            vals[r, :] = ex / (denv + 1e-16)

        plsc.parallel_loop(0, 128, 1, unroll=4)(edge_fn)
        pltpu.sync_copy(vals, attn_out.at[pl.ds(base0 + cidx * 128, 128)])

    def run_pass(base0, ta, tb, compute, with_snap):
        issue_idx(base0, 0, 0)
        wait_idx(0)
        issue_gathers(ta, tb, 0)
        issue_idx(base0, 1, 1)

        def pair_fn(t_, _):
            c0 = 2 * t_
            wait_idx(1)
            issue_gathers(ta, tb, 1)
            wait_gathers(ta, tb, 0)
            if with_snap:
                snap_dst(0)

            @pl.when(c0 + 2 < NCHUNK)
            def _pre0():
                issue_idx(base0, c0 + 2, 0)
            compute(base0, c0, 0)

            @pl.when(c0 + 2 < NCHUNK)
            def _g0():
                wait_idx(0)
                issue_gathers(ta, tb, 0)
            wait_gathers(ta, tb, 1)
            if with_snap:
                snap_dst(1)

            @pl.when(c0 + 3 < NCHUNK)
            def _pre1():
                issue_idx(base0, c0 + 3, 1)
            compute(base0, c0 + 1, 1)
            return _

        lax.fori_loop(0, NCHUNK // 2, pair_fn, None)

    pltpu.sync_copy(z16.at[pl.ds(0, arows)],
                    acc2.at[pl.ds(s * arows, arows)])
    plsc.subcore_barrier()
    for b in range(NB):
        run_pass(b * EP + w * TPW, tab2, tab2,
                 lambda base0, cidx, par: conv2_compute(par), True)
    plsc.subcore_barrier()
    pltpu.sync_copy(acc2.at[pl.ds(s * arows, arows)],
                    acc2_out.at[c, pl.ds(s * arows, arows)])
    run_pass(w * TPW, attab, attntab, attn_compute, False)


def _edge2_call(tab2_flat, attab_flat, attntab, srcg, dstg, z16):
    mesh = plsc.VectorSubcoreMesh(core_axis_name="c", subcore_axis_name="s")
    fn = pl.kernel(
        _edge2_body,
        out_type=[
            jax.ShapeDtypeStruct((2, NB * NP, 16), jnp.float32),
            jax.ShapeDtypeStruct((EP, 16), jnp.float32),
        ],
        mesh=mesh,
        compiler_params=pltpu.CompilerParams(use_tc_tiling_on_sc=False),
        scratch_types=(
            [pltpu.VMEM((128,), jnp.int32),
             pltpu.VMEM((128,), jnp.int32),
             pltpu.VMEM((128,), jnp.int32),
             pltpu.VMEM((128, 16), jnp.float32),
             pltpu.VMEM((128, 16), jnp.float32),
             pltpu.VMEM((128, 16), jnp.float32)] * 2
            + [pltpu.SemaphoreType.DMA] * 4
            + [pltpu.VMEM_SHARED((NB * NP, 16), jnp.float32)]
        ),
    )
    return fn(tab2_flat, attab_flat, attntab, srcg, dstg, z16)


def _pool_body(a20_ref, a21_ref, bb2_ref, batch_ref, ls_ref):
    A = a20_ref[0] + a21_ref[0]
    bcast = jnp.where(lax.broadcasted_iota(jnp.int32, (8, 8), 0) == 0, 1.0, 0.0)
    d = jnp.dot(A[:, 0:8], bcast, preferred_element_type=jnp.float32)
    x2 = A[:, 8:16] / (d + 1e-16) + bb2_ref[0, 0]
    x2 = jnp.where(x2 > 0.0, x2, jnp.exp(jnp.minimum(x2, 0.0)) - 1.0)
    gsel = lax.broadcasted_iota(jnp.int32, (1, 64), 1)
    oh = (batch_ref[0, 0].reshape(NP, 1) == gsel).astype(jnp.float32)
    ssum = lax.dot_general(oh, x2, (((0,), (0,)), ((), ())),
                           preferred_element_type=jnp.float32)
    cnt = jnp.sum(oh, axis=0).reshape(64, 1)
    pooled = ssum / jnp.maximum(cnt, 1.0)
    lane8 = lax.broadcasted_iota(jnp.int32, (64, 8), 1)
    mask = lane8 < 4
    mx = jnp.max(jnp.where(mask, pooled, -jnp.inf), axis=1, keepdims=True)
    se = jnp.sum(jnp.where(mask, jnp.exp(pooled - mx), 0.0),
                 axis=1, keepdims=True)
    ls_ref[0] = pooled - mx - jnp.log(se)


def _fuse_body(ls_ref, wf_ref, bf_ref, out_ref):
    c32 = jnp.concatenate([ls_ref[b] for b in range(NB)], axis=1)
    si = lax.broadcasted_iota(jnp.int32, (32, 16), 0)
    sj = lax.broadcasted_iota(jnp.int32, (32, 16), 1)
    S = jnp.where((sj == (si // 8) * 4 + (si % 8)) & (si % 8 < 4), 1.0, 0.0)
    xc = jnp.dot(c32, S, preferred_element_type=jnp.float32)
    res = jnp.dot(xc, wf_ref[...], preferred_element_type=jnp.float32) \
        + bf_ref[...]
    out_ref[...] = jnp.maximum(res[:, 0:4], 0.0)


def _final_call(a2_out, bb2_all, batch_all, wfp, bfp):
    ls = pl.pallas_call(
        _pool_body,
        grid=(NB,),
        in_specs=[
            pl.BlockSpec((1, NP, 16), lambda b: (b, 0, 0)),
            pl.BlockSpec((1, NP, 16), lambda b: (b, 0, 0)),
            pl.BlockSpec((1, 1, 8), lambda b: (b, 0, 0)),
            pl.BlockSpec((1, 1, NP), lambda b: (b, 0, 0)),
        ],
        out_specs=pl.BlockSpec((1, 64, 8), lambda b: (b, 0, 0)),
        out_shape=jax.ShapeDtypeStruct((NB, 64, 8), jnp.float32),
    )(a2_out[0].reshape(NB, NP, 16), a2_out[1].reshape(NB, NP, 16),
      bb2_all, batch_all)
    return pl.pallas_call(
        _fuse_body,
        out_shape=jax.ShapeDtypeStruct((64, 4), jnp.float32),
    )(ls, wfp, bfp)


def kernel(x_alpha, x_beta, x_theta, x_gamma, params,
           edge_index_alpha, edge_index_beta, edge_index_theta,
           edge_index_gamma, batch_alpha, batch_beta, batch_theta,
           batch_gamma):
    bands = ['alpha', 'beta', 'theta', 'gamma']
    xs = [x_alpha, x_beta, x_theta, x_gamma]
    eis = [edge_index_alpha, edge_index_beta, edge_index_theta,
           edge_index_gamma]
    bts = [batch_alpha, batch_beta, batch_theta, batch_gamma]

    x_all = jnp.stack(xs)
    g_all = jnp.stack([params[b]['bn1_g'] for b in bands]).reshape(NB, 1, F)
    b_all = jnp.stack([params[b]['bn1_b'] for b in bands]).reshape(NB, 1, F)
    w_all = jnp.stack([params[b]['conv1']['W'] for b in bands])
    asf_all = jnp.stack([params[b]['conv1']['as'].reshape(1, -1) for b in bands])
    adf_all = jnp.stack([params[b]['conv1']['ad'].reshape(1, -1) for b in bands])
    b1_all = jnp.stack([params[b]['conv1']['b'] for b in bands]).reshape(NB, 1, 64)
    g2_all = jnp.stack([params[b]['bn2_g'] for b in bands]).reshape(NB, 1, 64)
    b2_all = jnp.stack([params[b]['bn2_b'] for b in bands]).reshape(NB, 1, 64)
    w2p_all = jnp.stack([
        jnp.pad(params[b]['conv2']['W'], ((0, 0), (0, 4))) for b in bands])
    a2s_all = jnp.stack([
        jnp.pad(params[b]['conv2']['as'].reshape(-1), (0, 4)) for b in bands]).reshape(NB, 1, 8)
    a2d_all = jnp.stack([
        jnp.pad(params[b]['conv2']['ad'].reshape(-1), (0, 4)) for b in bands]).reshape(NB, 1, 8)
    bb2_all = jnp.stack([
        jnp.pad(params[b]['conv2']['b'], (0, 4)) for b in bands]).reshape(NB, 1, 8)
    wfp = jnp.pad(params['Wf'], ((0, 0), (0, 4)))
    bfp = jnp.pad(params['bf'], (0, 4))

    loop = jnp.arange(N, dtype=jnp.int32)
    padv = jnp.full((EP - E - N,), N, dtype=jnp.int32)
    srcg = jnp.concatenate([
        jnp.concatenate([ei[0], loop, padv]) + b * NP
        for b, ei in enumerate(eis)])
    dstg = jnp.concatenate([
        jnp.concatenate([ei[1], loop, padv]) + b * NP
        for b, ei in enumerate(eis)])
    batch_all = jnp.stack([
        jnp.concatenate([bt, jnp.full((NP - N,), 64, jnp.int32)])
        for bt in bts]).reshape(NB, 1, NP)

    z16 = jnp.zeros((NP, 16), jnp.float32)
    z64 = jnp.zeros((NP, 64), jnp.float32)

    attab_all, htab_all = _prep_call(x_all, g_all, b_all, w_all,
                                     asf_all, adf_all)
    attab_flat = attab_all.reshape(NB * NP, 16)
    htab_flat = htab_all.reshape(NB * NP, 64)

    den_out, agg_out = _edge1_call(attab_flat, htab_flat, srcg, dstg,
                                   z16, z64)

    tab2_all, attntab_all = _mid_call(den_out, agg_out, attab_all, b1_all,
                                      g2_all, b2_all, w2p_all,
                                      a2s_all, a2d_all)

    acc2_out, attn_wide = _edge2_call(tab2_all.reshape(NB * NP, 16),
                                      attab_flat, attntab_all[0],
                                      srcg, dstg, z16)

    out = _final_call(acc2_out, bb2_all, batch_all, wfp, bfp)
    attn = attn_wide[:E + N, 0:8]
    return out, attn
